# Initial kernel scaffold; baseline (speedup 1.0000x reference)
import jax
import jax.numpy as jnp
from jax.experimental import pallas as pl


def kernel(node_x, edge_x, t_m, edge_index, graph_ids, W_i, W_h, W_o_w, W_o_b):
    raise NotImplementedError("write your pallas kernel here")


# speedup vs baseline: 2.0215x; 2.0215x over previous
import functools

import jax
import jax.numpy as jnp
from jax import lax
from jax.experimental import pallas as pl
from jax.experimental.pallas import tpu as pltpu
from jax.experimental.pallas import tpu_sc as plsc

DEPTH = 3

NC = 2
NS = 16
NW = NC * NS

CHUNK = 80


def _init_body(ex_ref, t_ref, wi_ref, out_ref):
    binput = lax.dot_general(ex_ref[...], wi_ref[...],
                             (((1,), (1,)), ((), ())),
                             preferred_element_type=jnp.float32)
    out_ref[...] = jnp.maximum(binput, 0.0) + t_ref[...]


def _update_body(m_ref, c_ref, t_ref, ex_ref, wi_ref, wh_ref, out_ref):
    be = c_ref.shape[0]
    cm = c_ref[...]
    rows = lax.broadcasted_iota(jnp.int32, (be, cm.shape[1]), 0)
    even = (rows % 2) == 0
    rev = jnp.where(even, pltpu.roll(cm, be - 1, 0), pltpu.roll(cm, 1, 0))
    nei = m_ref[...] - rev
    binput = lax.dot_general(ex_ref[...], wi_ref[...],
                             (((1,), (1,)), ((), ())),
                             preferred_element_type=jnp.float32)
    upd = lax.dot_general(nei, wh_ref[...], (((1,), (1,)), ((), ())),
                          preferred_element_type=jnp.float32)
    out_ref[...] = jnp.maximum(binput + upd, 0.0) + t_ref[...]


def _add_body(a_ref, b_ref, out_ref):
    out_ref[...] = a_ref[...] + b_ref[...]


def _readout_body(nx_ref, s_ref, ids_ref, w1_ref, w2_ref, b_ref, out_ref,
                  sums_acc, cnt_acc, *, nblk, g):
    i = pl.program_id(0)

    @pl.when(i == 0)
    def _():
        sums_acc[...] = jnp.zeros_like(sums_acc)
        cnt_acc[...] = jnp.zeros_like(cnt_acc)

    bn = nx_ref.shape[0]
    ah = lax.dot_general(nx_ref[...], w1_ref[...], (((1,), (1,)), ((), ())),
                         preferred_element_type=jnp.float32)
    ah = ah + lax.dot_general(s_ref[...], w2_ref[...], (((1,), (1,)), ((), ())),
                              preferred_element_type=jnp.float32)
    ah = jnp.maximum(ah + b_ref[...], 0.0)

    ids = ids_ref[0]
    onehot = (lax.broadcasted_iota(jnp.int32, (g, bn), 0) == ids)
    onehot = onehot.astype(jnp.float32)
    sums_acc[...] += lax.dot_general(onehot, ah, (((1,), (0,)), ((), ())),
                                     preferred_element_type=jnp.float32)
    cnt_acc[...] += jnp.sum(onehot, axis=1, keepdims=True)

    @pl.when(i == nblk - 1)
    def _():
        out_ref[...] = sums_acc[...] / jnp.maximum(cnt_acc[...], 1.0)


ROWCHUNK = 400


def _sc_scatter_body(c_hbm, dst_hbm, zeros_hbm, out_hbm,
                     table, idx_v, rows_v, *, n, h, chunks_per_sub):
    cid = lax.axis_index("c")
    sid = lax.axis_index("s")
    nrow_chunks = n // ROWCHUNK

    for k in range((nrow_chunks + NS - 1) // NS):
        j = sid + NS * k

        @pl.when(j < nrow_chunks)
        def _():
            pltpu.sync_copy(zeros_hbm, table.at[pl.ds(j * ROWCHUNK, ROWCHUNK)])

    plsc.subcore_barrier()

    base0 = (cid * NS + sid) * chunks_per_sub

    @pl.loop(0, chunks_per_sub)
    def _(i):
        e0 = (base0 + i) * CHUNK
        pltpu.sync_copy(dst_hbm.at[pl.ds(e0, CHUNK)], idx_v)
        pltpu.sync_copy(c_hbm.at[pl.ds(e0, CHUNK)], rows_v)
        pltpu.sync_copy(rows_v, table.at[idx_v], add=True)

    plsc.subcore_barrier()
    for k in range((nrow_chunks + NS - 1) // NS):
        j = sid + NS * k

        @pl.when(j < nrow_chunks)
        def _():
            pltpu.sync_copy(table.at[pl.ds(j * ROWCHUNK, ROWCHUNK)],
                            out_hbm.at[cid, pl.ds(j * ROWCHUNK, ROWCHUNK)])


def _sc_gather_body(s_hbm, src_hbm, out_hbm, idx_v, rows_v, sem,
                    *, chunks_per_sub):
    cid = lax.axis_index("c")
    sid = lax.axis_index("s")
    base0 = (cid * NS + sid) * chunks_per_sub

    @pl.loop(0, chunks_per_sub)
    def _(i):
        e0 = (base0 + i) * CHUNK
        pltpu.sync_copy(src_hbm.at[pl.ds(e0, CHUNK)], idx_v)
        pltpu.async_copy(s_hbm.at[idx_v], rows_v, sem).wait()
        pltpu.sync_copy(rows_v, out_hbm.at[pl.ds(e0, CHUNK)])


@jax.jit
def _run(node_x, edge_x, t_m, edge_index, graph_ids, W_i, W_h, W_o_w, W_o_b):
    n, nf = node_x.shape
    e, ef = edge_x.shape
    h = t_m.shape[1]
    g = 256

    src = edge_index[0].astype(jnp.int32)
    dst = edge_index[1].astype(jnp.int32)
    ids = graph_ids.astype(jnp.int32)

    be = 512
    nblk_e = e // be
    bn = 1000
    nblk_n = n // bn

    chunks_per_sub = e // (CHUNK * NW)
    zeros_stripe = jnp.zeros((ROWCHUNK, h), jnp.float32)

    init = pl.pallas_call(
        _init_body,
        grid=(nblk_e,),
        in_specs=[pl.BlockSpec((be, ef), lambda i: (i, 0)),
                  pl.BlockSpec((be, h), lambda i: (i, 0)),
                  pl.BlockSpec((h, ef), lambda i: (0, 0))],
        out_specs=pl.BlockSpec((be, h), lambda i: (i, 0)),
        out_shape=jax.ShapeDtypeStruct((e, h), jnp.float32),
    )

    update = pl.pallas_call(
        _update_body,
        grid=(nblk_e,),
        in_specs=[pl.BlockSpec((be, h), lambda i: (i, 0)),
                  pl.BlockSpec((be, h), lambda i: (i, 0)),
                  pl.BlockSpec((be, h), lambda i: (i, 0)),
                  pl.BlockSpec((be, ef), lambda i: (i, 0)),
                  pl.BlockSpec((h, ef), lambda i: (0, 0)),
                  pl.BlockSpec((h, h), lambda i: (0, 0))],
        out_specs=pl.BlockSpec((be, h), lambda i: (i, 0)),
        out_shape=jax.ShapeDtypeStruct((e, h), jnp.float32),
    )

    add2 = pl.pallas_call(
        _add_body,
        grid=(nblk_n,),
        in_specs=[pl.BlockSpec((bn, h), lambda i: (i, 0)),
                  pl.BlockSpec((bn, h), lambda i: (i, 0))],
        out_specs=pl.BlockSpec((bn, h), lambda i: (i, 0)),
        out_shape=jax.ShapeDtypeStruct((n, h), jnp.float32),
    )

    mesh = plsc.VectorSubcoreMesh(core_axis_name="c", subcore_axis_name="s",
                                  num_cores=NC, num_subcores=NS)

    scatter = pl.kernel(
        functools.partial(_sc_scatter_body, n=n, h=h,
                          chunks_per_sub=chunks_per_sub),
        out_type=jax.ShapeDtypeStruct((NC, n, h), jnp.float32),
        mesh=mesh,
        scratch_types=[pltpu.VMEM_SHARED((n, h), jnp.float32),
                       pltpu.VMEM((CHUNK,), jnp.int32),
                       pltpu.VMEM((CHUNK, h), jnp.float32)],
    )

    gather = pl.kernel(
        functools.partial(_sc_gather_body, chunks_per_sub=chunks_per_sub),
        out_type=jax.ShapeDtypeStruct((e, h), jnp.float32),
        mesh=mesh,
        scratch_types=[pltpu.VMEM((CHUNK,), jnp.int32),
                       pltpu.VMEM((CHUNK, h), jnp.float32),
                       pltpu.SemaphoreType.DMA],
    )

    readout = pl.pallas_call(
        functools.partial(_readout_body, nblk=nblk_n, g=g),
        grid=(nblk_n,),
        in_specs=[pl.BlockSpec((bn, nf), lambda i: (i, 0)),
                  pl.BlockSpec((bn, h), lambda i: (i, 0)),
                  pl.BlockSpec((1, 1, bn), lambda i: (i, 0, 0)),
                  pl.BlockSpec((h, nf), lambda i: (0, 0)),
                  pl.BlockSpec((h, h), lambda i: (0, 0)),
                  pl.BlockSpec((1, h), lambda i: (0, 0))],
        out_specs=pl.BlockSpec((g, h), lambda i: (0, 0)),
        out_shape=jax.ShapeDtypeStruct((g, h), jnp.float32),
        scratch_shapes=[pltpu.VMEM((g, h), jnp.float32),
                        pltpu.VMEM((g, 1), jnp.float32)],
    )

    c = init(edge_x, t_m, W_i)
    for _ in range(DEPTH - 1):
        parts = scatter(c, dst, zeros_stripe)
        s = add2(parts[0], parts[1])
        m = gather(s, src)
        c = update(m, c, t_m, edge_x, W_i, W_h)
    parts = scatter(c, dst, zeros_stripe)
    s = add2(parts[0], parts[1])

    w1 = W_o_w[:, :nf]
    w2 = W_o_w[:, nf:]
    ids3 = ids.reshape(nblk_n, 1, bn)
    return readout(node_x, s, ids3, w1, w2, W_o_b.reshape(1, h))


def kernel(node_x, edge_x, t_m, edge_index, graph_ids, W_i, W_h, W_o_w, W_o_b):
    return _run(node_x, edge_x, t_m, edge_index, graph_ids,
                W_i, W_h, W_o_w, W_o_b)


# speedup vs baseline: 3.9637x; 1.9608x over previous
import functools

import jax
import jax.numpy as jnp
from jax import lax
from jax.experimental import pallas as pl
from jax.experimental.pallas import tpu as pltpu
from jax.experimental.pallas import tpu_sc as plsc

DEPTH = 3

NC = 2
NS = 16
NW = NC * NS

CHUNK = 80
ROWCHUNK = 400


def _init_body(ex_ref, t_ref, wi_ref, out_ref):
    binput = lax.dot_general(ex_ref[...], wi_ref[...],
                             (((1,), (1,)), ((), ())),
                             preferred_element_type=jnp.float32)
    out_ref[...] = jnp.maximum(binput, 0.0) + t_ref[...]


def _update_body(m_ref, c_ref, t_ref, ex_ref, wi_ref, wh_ref, out_ref):
    be = c_ref.shape[0]
    cm = c_ref[...]
    rows = lax.broadcasted_iota(jnp.int32, (be, cm.shape[1]), 0)
    even = (rows % 2) == 0
    rev = jnp.where(even, pltpu.roll(cm, be - 1, 0), pltpu.roll(cm, 1, 0))
    nei = m_ref[...] - rev
    binput = lax.dot_general(ex_ref[...], wi_ref[...],
                             (((1,), (1,)), ((), ())),
                             preferred_element_type=jnp.float32)
    upd = lax.dot_general(nei, wh_ref[...], (((1,), (1,)), ((), ())),
                          preferred_element_type=jnp.float32)
    out_ref[...] = jnp.maximum(binput + upd, 0.0) + t_ref[...]


def _add_body(a_ref, b_ref, out_ref):
    out_ref[...] = a_ref[...] + b_ref[...]


def _readout_body(nx_ref, s_ref, ids_ref, w1_ref, w2_ref, b_ref, out_ref,
                  sums_acc, cnt_acc, *, nblk, g):
    i = pl.program_id(0)

    @pl.when(i == 0)
    def _():
        sums_acc[...] = jnp.zeros_like(sums_acc)
        cnt_acc[...] = jnp.zeros_like(cnt_acc)

    bn = nx_ref.shape[0]
    ah = lax.dot_general(nx_ref[...], w1_ref[...], (((1,), (1,)), ((), ())),
                         preferred_element_type=jnp.float32)
    ah = ah + lax.dot_general(s_ref[...], w2_ref[...], (((1,), (1,)), ((), ())),
                              preferred_element_type=jnp.float32)
    ah = jnp.maximum(ah + b_ref[...], 0.0)

    ids = ids_ref[0]
    onehot = (lax.broadcasted_iota(jnp.int32, (g, bn), 0) == ids)
    onehot = onehot.astype(jnp.float32)
    sums_acc[...] += lax.dot_general(onehot, ah, (((1,), (0,)), ((), ())),
                                     preferred_element_type=jnp.float32)
    cnt_acc[...] += jnp.sum(onehot, axis=1, keepdims=True)

    @pl.when(i == nblk - 1)
    def _():
        out_ref[...] = sums_acc[...] / jnp.maximum(cnt_acc[...], 1.0)


def _sc_scatter_body(c_hbm, dst3_hbm, zeros_hbm, out_hbm,
                     table, idx_all, rows_a, rows_b, sem_a, sem_b,
                     *, n, h, chunks_per_sub):
    cid = lax.axis_index("c")
    sid = lax.axis_index("s")
    wid = cid * NS + sid
    nrow_chunks = n // ROWCHUNK

    for k in range((nrow_chunks + NS - 1) // NS):
        j = sid + NS * k

        @pl.when(j < nrow_chunks)
        def _():
            pltpu.sync_copy(zeros_hbm, table.at[pl.ds(j * ROWCHUNK, ROWCHUNK)])

    idx_dma = pltpu.async_copy(dst3_hbm.at[wid], idx_all, sem_a)
    base0 = wid * chunks_per_sub
    bufs = ((rows_a, sem_a), (rows_b, sem_b))

    def load(ci, rows_v, sem):
        pltpu.async_copy(c_hbm.at[pl.ds((base0 + ci) * CHUNK, CHUNK)],
                         rows_v, sem)

    def wait_load(ci, rows_v, sem):
        pltpu.make_async_copy(c_hbm.at[pl.ds((base0 + ci) * CHUNK, CHUNK)],
                              rows_v, sem).wait()

    idx_dma.wait()
    plsc.subcore_barrier()

    load(0, rows_a, sem_a)
    load(1, rows_b, sem_b)

    npairs = chunks_per_sub // 2
    rem = chunks_per_sub - 2 * npairs

    @pl.loop(0, npairs)
    def _(g):
        for k in range(2):
            rows_v, sem = bufs[k]
            ci = 2 * g + k
            wait_load(ci, rows_v, sem)
            pltpu.sync_copy(rows_v, table.at[idx_all.at[ci]], add=True)

            @pl.when(ci + 2 < chunks_per_sub)
            def _():
                load(ci + 2, rows_v, sem)

    if rem:
        ci = chunks_per_sub - 1
        rows_v, sem = bufs[ci % 2]
        wait_load(ci, rows_v, sem)
        pltpu.sync_copy(rows_v, table.at[idx_all.at[ci]], add=True)

    plsc.subcore_barrier()
    for k in range((nrow_chunks + NS - 1) // NS):
        j = sid + NS * k

        @pl.when(j < nrow_chunks)
        def _():
            pltpu.sync_copy(table.at[pl.ds(j * ROWCHUNK, ROWCHUNK)],
                            out_hbm.at[cid, pl.ds(j * ROWCHUNK, ROWCHUNK)])


def _sc_gather_body(s_hbm, src3_hbm, out_hbm,
                    idx_all, rows_a, rows_b, rows_c,
                    sem_i, sem_ga, sem_gb, sem_gc, sem_sa, sem_sb, sem_sc,
                    *, chunks_per_sub):
    cid = lax.axis_index("c")
    sid = lax.axis_index("s")
    wid = cid * NS + sid
    base0 = wid * chunks_per_sub

    gbufs = ((rows_a, sem_ga, sem_sa), (rows_b, sem_gb, sem_sb),
             (rows_c, sem_gc, sem_sc))

    pltpu.async_copy(src3_hbm.at[wid], idx_all, sem_i).wait()

    def issue_gather(ci, rows_v, sem_g):
        pltpu.async_copy(s_hbm.at[idx_all.at[ci]], rows_v, sem_g)

    def wait_gather(ci, rows_v, sem_g):
        pltpu.make_async_copy(s_hbm.at[idx_all.at[ci]], rows_v, sem_g).wait()

    def issue_store(ci, rows_v, sem_s):
        pltpu.async_copy(rows_v,
                         out_hbm.at[pl.ds((base0 + ci) * CHUNK, CHUNK)], sem_s)

    def wait_store(ci, rows_v, sem_s):
        pltpu.make_async_copy(
            rows_v, out_hbm.at[pl.ds((base0 + ci) * CHUNK, CHUNK)],
            sem_s).wait()

    issue_gather(0, rows_a, sem_ga)
    issue_gather(1, rows_b, sem_gb)

    ntrip = chunks_per_sub // 3
    nch = chunks_per_sub

    @pl.loop(0, ntrip)
    def _(g):
        for k in range(3):
            ci = 3 * g + k
            rows_v, sem_g, sem_s = gbufs[k]
            wait_gather(ci, rows_v, sem_g)
            issue_store(ci, rows_v, sem_s)
            k2 = (k + 2) % 3
            rows2, sem_g2, sem_s2 = gbufs[k2]

            @pl.when(ci + 2 < nch)
            def _():
                @pl.when(ci - 1 >= 0)
                def _():
                    wait_store(ci - 1, rows2, sem_s2)
                issue_gather(ci + 2, rows2, sem_g2)

    for ci in range(3 * ntrip, nch):
        rows_v, sem_g, sem_s = gbufs[ci % 3]
        wait_gather(ci, rows_v, sem_g)
        issue_store(ci, rows_v, sem_s)

    for ci in range(nch - 3, nch):
        rows_v, sem_g, sem_s = gbufs[ci % 3]
        wait_store(ci, rows_v, sem_s)


@jax.jit
def _run(node_x, edge_x, t_m, edge_index, graph_ids, W_i, W_h, W_o_w, W_o_b):
    n, nf = node_x.shape
    e, ef = edge_x.shape
    h = t_m.shape[1]
    g = 256

    src = edge_index[0].astype(jnp.int32)
    dst = edge_index[1].astype(jnp.int32)
    ids = graph_ids.astype(jnp.int32)

    be = 2000
    nblk_e = e // be
    bn = 2000
    nblk_n = n // bn

    chunks_per_sub = e // (CHUNK * NW)
    src3 = src.reshape(NW, chunks_per_sub, CHUNK)
    dst3 = dst.reshape(NW, chunks_per_sub, CHUNK)
    zeros_stripe = jnp.zeros((ROWCHUNK, h), jnp.float32)

    init = pl.pallas_call(
        _init_body,
        grid=(nblk_e,),
        in_specs=[pl.BlockSpec((be, ef), lambda i: (i, 0)),
                  pl.BlockSpec((be, h), lambda i: (i, 0)),
                  pl.BlockSpec((h, ef), lambda i: (0, 0))],
        out_specs=pl.BlockSpec((be, h), lambda i: (i, 0)),
        out_shape=jax.ShapeDtypeStruct((e, h), jnp.float32),
    )

    update = pl.pallas_call(
        _update_body,
        grid=(nblk_e,),
        in_specs=[pl.BlockSpec((be, h), lambda i: (i, 0)),
                  pl.BlockSpec((be, h), lambda i: (i, 0)),
                  pl.BlockSpec((be, h), lambda i: (i, 0)),
                  pl.BlockSpec((be, ef), lambda i: (i, 0)),
                  pl.BlockSpec((h, ef), lambda i: (0, 0)),
                  pl.BlockSpec((h, h), lambda i: (0, 0))],
        out_specs=pl.BlockSpec((be, h), lambda i: (i, 0)),
        out_shape=jax.ShapeDtypeStruct((e, h), jnp.float32),
    )

    add2 = pl.pallas_call(
        _add_body,
        grid=(nblk_n,),
        in_specs=[pl.BlockSpec((bn, h), lambda i: (i, 0)),
                  pl.BlockSpec((bn, h), lambda i: (i, 0))],
        out_specs=pl.BlockSpec((bn, h), lambda i: (i, 0)),
        out_shape=jax.ShapeDtypeStruct((n, h), jnp.float32),
    )

    mesh = plsc.VectorSubcoreMesh(core_axis_name="c", subcore_axis_name="s",
                                  num_cores=NC, num_subcores=NS)

    scatter = pl.kernel(
        functools.partial(_sc_scatter_body, n=n, h=h,
                          chunks_per_sub=chunks_per_sub),
        out_type=jax.ShapeDtypeStruct((NC, n, h), jnp.float32),
        mesh=mesh,
        scratch_types=[pltpu.VMEM_SHARED((n, h), jnp.float32),
                       pltpu.VMEM((chunks_per_sub, CHUNK), jnp.int32),
                       pltpu.VMEM((CHUNK, h), jnp.float32),
                       pltpu.VMEM((CHUNK, h), jnp.float32),
                       pltpu.SemaphoreType.DMA,
                       pltpu.SemaphoreType.DMA],
    )

    gather = pl.kernel(
        functools.partial(_sc_gather_body, chunks_per_sub=chunks_per_sub),
        out_type=jax.ShapeDtypeStruct((e, h), jnp.float32),
        mesh=mesh,
        scratch_types=[pltpu.VMEM((chunks_per_sub, CHUNK), jnp.int32),
                       pltpu.VMEM((CHUNK, h), jnp.float32),
                       pltpu.VMEM((CHUNK, h), jnp.float32),
                       pltpu.VMEM((CHUNK, h), jnp.float32),
                       pltpu.SemaphoreType.DMA,
                       pltpu.SemaphoreType.DMA,
                       pltpu.SemaphoreType.DMA,
                       pltpu.SemaphoreType.DMA,
                       pltpu.SemaphoreType.DMA,
                       pltpu.SemaphoreType.DMA,
                       pltpu.SemaphoreType.DMA],
    )

    readout = pl.pallas_call(
        functools.partial(_readout_body, nblk=nblk_n, g=g),
        grid=(nblk_n,),
        in_specs=[pl.BlockSpec((bn, nf), lambda i: (i, 0)),
                  pl.BlockSpec((bn, h), lambda i: (i, 0)),
                  pl.BlockSpec((1, 1, bn), lambda i: (i, 0, 0)),
                  pl.BlockSpec((h, nf), lambda i: (0, 0)),
                  pl.BlockSpec((h, h), lambda i: (0, 0)),
                  pl.BlockSpec((1, h), lambda i: (0, 0))],
        out_specs=pl.BlockSpec((g, h), lambda i: (0, 0)),
        out_shape=jax.ShapeDtypeStruct((g, h), jnp.float32),
        scratch_shapes=[pltpu.VMEM((g, h), jnp.float32),
                        pltpu.VMEM((g, 1), jnp.float32)],
    )

    c = init(edge_x, t_m, W_i)
    for _ in range(DEPTH - 1):
        parts = scatter(c, dst3, zeros_stripe)
        s = add2(parts[0], parts[1])
        m = gather(s, src3)
        c = update(m, c, t_m, edge_x, W_i, W_h)
    parts = scatter(c, dst3, zeros_stripe)
    s = add2(parts[0], parts[1])

    w1 = W_o_w[:, :nf]
    w2 = W_o_w[:, nf:]
    ids3 = ids.reshape(nblk_n, 1, bn)
    return readout(node_x, s, ids3, w1, w2, W_o_b.reshape(1, h))


def kernel(node_x, edge_x, t_m, edge_index, graph_ids, W_i, W_h, W_o_w, W_o_b):
    return _run(node_x, edge_x, t_m, edge_index, graph_ids,
                W_i, W_h, W_o_w, W_o_b)
